# Initial kernel scaffold; baseline (speedup 1.0000x reference)
import jax
import jax.numpy as jnp
from jax.experimental import pallas as pl


def kernel(pred_logits, pred_boxes, pred_super_logits, target_sizes):
    raise NotImplementedError("write your pallas kernel here")


# speedup vs baseline: 6.2355x; 6.2355x over previous
import functools
import jax
import jax.numpy as jnp
import numpy as np
from jax.experimental import pallas as pl
from jax.experimental.pallas import tpu as pltpu

_B, _Q, _C, _S = 8, 20000, 91, 12
_K = 100
_NEG = -100000000000.0
_TQ = 1000

_SUP_ONEHOT = np.zeros((128, _C), np.float32)
_SUP_ONEHOT[(np.arange(_C) % _S), np.arange(_C)] = 1.0


def _stage1_body(l_ref, s_ref, g_ref, prob_ref, m_ref):
    l = l_ref[...]
    s = s_ref[...]
    lcol = jax.lax.broadcasted_iota(jnp.int32, (_TQ, _C), 1)
    scol = jax.lax.broadcasted_iota(jnp.int32, (_TQ, _S), 1)
    sigl = jax.nn.sigmoid(jnp.where(lcol == _C - 1, _NEG, l))
    sigs = jax.nn.sigmoid(jnp.where(scol == _S - 1, _NEG, s))
    sigs_z = jnp.concatenate(
        [sigs, jnp.zeros((_TQ, 128 - _S), jnp.float32)], axis=1)
    supg = jnp.dot(sigs_z, g_ref[...], preferred_element_type=jnp.float32,
                   precision=jax.lax.Precision.HIGHEST)
    prob = sigl * supg
    prob_ref[...] = prob
    m_ref[...] = jnp.max(prob, axis=1, keepdims=True)


def _topq_body(m_ref, idx_ref, mv_ref):
    mv_ref[...] = m_ref[...]
    col = jax.lax.broadcasted_iota(jnp.int32, (_B, _Q), 1)
    kcol = jax.lax.broadcasted_iota(jnp.int32, (_B, _K), 1)

    def body(k, acc):
        mv = mv_ref[...]
        cur = jnp.max(mv, axis=1, keepdims=True)
        amx = jnp.min(jnp.where(mv == cur, col, _Q), axis=1, keepdims=True)
        acc = jnp.where(kcol == k, amx, acc)
        mv_ref[...] = jnp.where(col == amx, -jnp.inf, mv)
        return acc

    idx_ref[...] = jax.lax.fori_loop(0, _K, body, jnp.zeros((_B, _K), jnp.int32))


def _final_body(v_ref, f_ref, bx_ref, ts_ref, sc_ref, lb_ref, bo_ref, vv_ref):
    n = _K * _C
    vv_ref[...] = v_ref[...]
    f = f_ref[...]
    col = jax.lax.broadcasted_iota(jnp.int32, (_B, n), 1)
    rowof = col // _C
    krow = jax.lax.broadcasted_iota(jnp.int32, (_B, _K), 1)
    kcol = krow

    cx = bx_ref[:, :, 0]
    cy = bx_ref[:, :, 1]
    w = bx_ref[:, :, 2]
    h = bx_ref[:, :, 3]
    img_h = ts_ref[:, 0:1]
    img_w = ts_ref[:, 1:2]
    x0 = (cx - 0.5 * w) * img_w
    y0 = (cy - 0.5 * h) * img_h
    x1 = (cx + 0.5 * w) * img_w
    y1 = (cy + 0.5 * h) * img_h

    big = jnp.int32(1 << 30)
    zf = jnp.zeros((_B, _K), jnp.float32)

    def body(k, acc):
        asc, alb, ax0, ay0, ax1, ay1 = acc
        vv = vv_ref[...]
        cur = jnp.max(vv, axis=1, keepdims=True)
        hit = vv == cur
        sel_f = jnp.min(jnp.where(hit, f, big), axis=1, keepdims=True)
        selmask = f == sel_f
        rowsel = jnp.min(jnp.where(selmask, rowof, big), axis=1, keepdims=True)
        at_k = kcol == k
        asc = jnp.where(at_k, cur, asc)
        alb = jnp.where(at_k, sel_f % _C, alb)
        oh = (krow == rowsel).astype(jnp.float32)
        ax0 = jnp.where(at_k, jnp.sum(x0 * oh, axis=1, keepdims=True), ax0)
        ay0 = jnp.where(at_k, jnp.sum(y0 * oh, axis=1, keepdims=True), ay0)
        ax1 = jnp.where(at_k, jnp.sum(x1 * oh, axis=1, keepdims=True), ax1)
        ay1 = jnp.where(at_k, jnp.sum(y1 * oh, axis=1, keepdims=True), ay1)
        vv_ref[...] = jnp.where(selmask, -jnp.inf, vv)
        return (asc, alb, ax0, ay0, ax1, ay1)

    asc, alb, ax0, ay0, ax1, ay1 = jax.lax.fori_loop(
        0, _K, body, (zf, jnp.zeros((_B, _K), jnp.int32), zf, zf, zf, zf))
    sc_ref[...] = asc
    lb_ref[...] = alb
    bo_ref[:, :, 0:1] = ax0[:, :, None]
    bo_ref[:, :, 1:2] = ay0[:, :, None]
    bo_ref[:, :, 2:3] = ax1[:, :, None]
    bo_ref[:, :, 3:4] = ay1[:, :, None]


def kernel(pred_logits, pred_boxes, pred_super_logits, target_sizes):
    nrows = _B * _Q
    l2 = pred_logits.reshape(nrows, _C)
    s2 = pred_super_logits.reshape(nrows, _S)
    onehot = jnp.asarray(_SUP_ONEHOT)

    prob, m = pl.pallas_call(
        _stage1_body,
        grid=(nrows // _TQ,),
        in_specs=[
            pl.BlockSpec((_TQ, _C), lambda i: (i, 0)),
            pl.BlockSpec((_TQ, _S), lambda i: (i, 0)),
            pl.BlockSpec((128, _C), lambda i: (0, 0)),
        ],
        out_specs=[
            pl.BlockSpec((_TQ, _C), lambda i: (i, 0)),
            pl.BlockSpec((_TQ, 1), lambda i: (i, 0)),
        ],
        out_shape=[
            jax.ShapeDtypeStruct((nrows, _C), jnp.float32),
            jax.ShapeDtypeStruct((nrows, 1), jnp.float32),
        ],
    )(l2, s2, onehot)

    mview = m.reshape(_B, _Q)
    cand = pl.pallas_call(
        _topq_body,
        out_shape=jax.ShapeDtypeStruct((_B, _K), jnp.int32),
        scratch_shapes=[pltpu.VMEM((_B, _Q), jnp.float32)],
    )(mview)

    probc = jnp.take_along_axis(
        prob.reshape(_B, _Q, _C), cand[:, :, None], axis=1)
    boxc = jnp.take_along_axis(pred_boxes, cand[:, :, None], axis=1)
    flatf = (cand[:, :, None] * _C +
             jnp.arange(_C, dtype=jnp.int32)[None, None, :])

    scores, labels, boxes = pl.pallas_call(
        _final_body,
        out_shape=[
            jax.ShapeDtypeStruct((_B, _K), jnp.float32),
            jax.ShapeDtypeStruct((_B, _K), jnp.int32),
            jax.ShapeDtypeStruct((_B, _K, 4), jnp.float32),
        ],
        scratch_shapes=[pltpu.VMEM((_B, _K * _C), jnp.float32)],
    )(probc.reshape(_B, _K * _C), flatf.reshape(_B, _K * _C), boxc,
      target_sizes.astype(jnp.float32))

    return scores, labels, boxes
